# baseline (speedup 1.0000x reference)
import functools

import jax
import jax.numpy as jnp
from jax import lax
from jax.experimental import pallas as pl
from jax.experimental.pallas import tpu as pltpu
from jax.experimental.pallas import tpu_sc as plsc

D = 16
F = 26
BROWS = 4096
NROWS = 1000000
NC, NS = 2, 16
NW = NC * NS
RPW = BROWS // NW
CHUNK = 128
NCHUNK = RPW * F // CHUNK
NEG_W = 0.5
CBLK = 16384


def _conv_body(x_ref, o_ref):
    x = x_ref[...]
    y = jnp.transpose(x, (1, 0))
    y3 = y.reshape(CBLK // 8, 8, 16)
    o_ref[...] = jnp.concatenate([y3[:, t, :] for t in range(8)], axis=1)


def _relayout(table):
    t = table.T
    grid = (NROWS + CBLK - 1) // CBLK
    conv = pl.pallas_call(
        _conv_body,
        grid=(grid,),
        in_specs=[pl.BlockSpec((16, CBLK), lambda m: (0, m))],
        out_specs=pl.BlockSpec((CBLK // 8, 128), lambda m: (m, 0)),
        out_shape=jax.ShapeDtypeStruct((NROWS * 16 // 128, 128), jnp.float32),
    )(t)
    return conv.reshape(NROWS, D)


def _sc_reduce_body(u_table, i_table, w_u, w_i, u_idx, i_idx, u_widx, i_widx,
                    u_sum, u_sq, u_w, i_sum, i_sq, i_w,
                    idx_v, widx_v, emb_v, wg_v, sum_v, sq_v, wsum_v,
                    sem_e, sem_w):
    wid = lax.axis_index("s") * NC + lax.axis_index("c")
    base = wid * RPW

    for (table, wtab, idx_hbm, widx_hbm, o_sum, o_sq, o_w) in (
        (u_table, w_u, u_idx, u_widx, u_sum, u_sq, u_w),
        (i_table, w_i, i_idx, i_widx, i_sum, i_sq, i_w),
    ):
        pltpu.sync_copy(idx_hbm.at[pl.ds(wid * RPW * F, RPW * F)], idx_v)
        pltpu.sync_copy(widx_hbm.at[pl.ds(wid * RPW * F, RPW * F)], widx_v)
        copies = []
        for j in range(NCHUNK):
            copies.append(pltpu.async_copy(
                table.at[idx_v.at[pl.ds(j * CHUNK, CHUNK)]],
                emb_v.at[pl.ds(j * CHUNK, CHUNK)], sem_e))
            copies.append(pltpu.async_copy(
                wtab.at[widx_v.at[pl.ds(j * CHUNK, CHUNK)]],
                wg_v.at[pl.ds(j * CHUNK, CHUNK)], sem_w))
        for c in copies:
            c.wait()

        def row_body(b, _):
            k0 = b * F
            acc = emb_v[k0, :]
            sq = acc * acc
            for f in range(1, F):
                v = emb_v[k0 + f, :]
                acc = acc + v
                sq = sq + v * v
            sum_v[b, :] = acc
            sq_v[b, :] = sq
            return 0

        lax.fori_loop(0, RPW, row_body, 0, unroll=False)

        def w_body(g, _):
            k0 = g * (16 * F)
            acc = wg_v[pl.ds(k0, 16)]
            for f in range(1, F):
                acc = acc + wg_v[pl.ds(k0 + f * 16, 16)]
            wsum_v[pl.ds(g * 16, 16)] = acc
            return 0

        lax.fori_loop(0, RPW // 16, w_body, 0, unroll=False)

        pltpu.sync_copy(sum_v, o_sum.at[pl.ds(base, RPW)])
        pltpu.sync_copy(sq_v, o_sq.at[pl.ds(base, RPW)])
        pltpu.sync_copy(wsum_v, o_w.at[pl.ds(base, RPW)])


def _sc_pair_body(u_sum, u_sq, u_w, i_sum, i_sq, i_w, a_idx, b_idx,
                  us_o, uq_o, uw_o, is_o, iq_o, iw_o,
                  av, bv, rsum, rsq, rw, sem):
    wid = lax.axis_index("s") * NC + lax.axis_index("c")
    base = wid * RPW
    pltpu.sync_copy(a_idx.at[pl.ds(base, RPW)], av)
    pltpu.sync_copy(b_idx.at[pl.ds(base, RPW)], bv)
    for (src_sum, src_sq, src_w, iv, o_sum, o_sq, o_w) in (
        (u_sum, u_sq, u_w, av, us_o, uq_o, uw_o),
        (i_sum, i_sq, i_w, bv, is_o, iq_o, iw_o),
    ):
        pltpu.async_copy(src_sum.at[iv], rsum, sem).wait()
        pltpu.sync_copy(rsum, o_sum.at[pl.ds(base, RPW)])
        pltpu.async_copy(src_sq.at[iv], rsq, sem).wait()
        pltpu.sync_copy(rsq, o_sq.at[pl.ds(base, RPW)])
        pltpu.async_copy(src_w.at[iv], rw, sem).wait()
        pltpu.sync_copy(rw, o_w.at[pl.ds(base, RPW)])


def _tc_final_body(usum_ref, isum_ref, us_ref, uq_ref, uw_ref,
                   is_ref, iq_ref, iw_ref, h1_ref, h2_ref, gb_ref, out_ref):
    P = usum_ref[...]
    Q = isum_ref[...]
    dn = (((0,), (0,)), ((), ()))
    PtP = lax.dot_general(P, P, dn, precision=lax.Precision.HIGHEST,
                          preferred_element_type=jnp.float32)
    QtQ = lax.dot_general(Q, Q, dn, precision=lax.Precision.HIGHEST,
                          preferred_element_type=jnp.float32)
    h2r = h2_ref[...]
    M = PtP * QtQ
    t = lax.dot_general(h2r, M, (((1,), (0,)), ((), ())),
                        precision=lax.Precision.HIGHEST,
                        preferred_element_type=jnp.float32)
    whole = NEG_W * jnp.sum(t * h2r)

    h1r = h1_ref[...]
    us = us_ref[...]
    vs = is_ref[...]
    bi_u = 0.5 * (us * us - uq_ref[...])
    bi_v = 0.5 * (vs * vs - iq_ref[...])
    s_u = jnp.sum(bi_u * h1r, axis=1, keepdims=True) + uw_ref[...] + gb_ref[0, 0]
    s_v = jnp.sum(bi_v * h1r, axis=1, keepdims=True) + iw_ref[...]
    y = jnp.sum(us * vs * h2r, axis=1, keepdims=True) + s_u + s_v
    pair = jnp.sum((1.0 - NEG_W) * y * y - 2.0 * y)
    out_ref[...] = jnp.reshape(whole + pair, (1, 1))


def kernel(user_features_embeddings, item_features_embeddings, w_user, w_item,
           global_bias, h1, h2, user_feature_indexes, item_feature_indexes,
           positive_pairs):
    f32 = jnp.float32
    u_idx2 = user_feature_indexes.astype(jnp.int32).reshape(-1)
    i_idx2 = item_feature_indexes.astype(jnp.int32).reshape(-1)
    u_widx = (user_feature_indexes.astype(jnp.int32)
              .reshape(BROWS // 16, 16, F).transpose(0, 2, 1).reshape(-1))
    i_widx = (item_feature_indexes.astype(jnp.int32)
              .reshape(BROWS // 16, 16, F).transpose(0, 2, 1).reshape(-1))
    a_idx = positive_pairs[:, 0].astype(jnp.int32)
    b_idx = positive_pairs[:, 1].astype(jnp.int32)

    conv_u = _relayout(user_features_embeddings)
    conv_i = item_features_embeddings

    mesh = plsc.VectorSubcoreMesh(core_axis_name="c", subcore_axis_name="s")
    stat = jax.ShapeDtypeStruct((BROWS, D), f32)
    vec = jax.ShapeDtypeStruct((BROWS,), f32)

    sc_params = pltpu.CompilerParams(use_tc_tiling_on_sc=False)
    reduce_fn = pl.kernel(
        _sc_reduce_body,
        out_type=(stat, stat, vec, stat, stat, vec),
        mesh=mesh,
        compiler_params=sc_params,
        scratch_types=[
            pltpu.VMEM((RPW * F,), jnp.int32),
            pltpu.VMEM((RPW * F,), jnp.int32),
            pltpu.VMEM((RPW * F, D), f32),
            pltpu.VMEM((RPW * F,), f32),
            pltpu.VMEM((RPW, D), f32),
            pltpu.VMEM((RPW, D), f32),
            pltpu.VMEM((RPW,), f32),
            pltpu.SemaphoreType.DMA,
            pltpu.SemaphoreType.DMA,
        ],
    )
    u_sum, u_sq, u_w, i_sum, i_sq, i_w = reduce_fn(
        conv_u, conv_i, w_user, w_item, u_idx2, i_idx2, u_widx, i_widx)

    pair_fn = pl.kernel(
        _sc_pair_body,
        out_type=(stat, stat, vec, stat, stat, vec),
        mesh=mesh,
        compiler_params=sc_params,
        scratch_types=[
            pltpu.VMEM((RPW,), jnp.int32),
            pltpu.VMEM((RPW,), jnp.int32),
            pltpu.VMEM((RPW, D), f32),
            pltpu.VMEM((RPW, D), f32),
            pltpu.VMEM((RPW,), f32),
            pltpu.SemaphoreType.DMA,
        ],
    )
    us, uq, uw, is_, iq, iw = pair_fn(u_sum, u_sq, u_w, i_sum, i_sq, i_w,
                                      a_idx, b_idx)

    loss = pl.pallas_call(
        _tc_final_body,
        out_shape=jax.ShapeDtypeStruct((1, 1), f32),
    )(u_sum, i_sum, us, uq, uw.reshape(BROWS, 1),
      is_, iq, iw.reshape(BROWS, 1),
      h1.reshape(1, D), h2.reshape(1, D), global_bias.reshape(1, 1))
    return loss.reshape(1)


# speedup vs baseline: 1.0127x; 1.0127x over previous
import functools

import jax
import jax.numpy as jnp
from jax import lax
from jax.experimental import pallas as pl
from jax.experimental.pallas import tpu as pltpu
from jax.experimental.pallas import tpu_sc as plsc

D = 16
F = 26
BROWS = 4096
NROWS = 1000000
NC, NS = 2, 16
NW = NC * NS
RPW = BROWS // NW
CHUNK = 128
NCHUNK = RPW * F // CHUNK
NEG_W = 0.5
CBLK = 8192


def _conv_body(x_ref, o_ref):
    x = x_ref[...]
    y = jnp.transpose(x, (1, 0))
    y3 = y.reshape(CBLK // 8, 8, 16)
    o_ref[...] = jnp.concatenate([y3[:, t, :] for t in range(8)], axis=1)


def _relayout(table):
    t = table.T
    grid = (NROWS + CBLK - 1) // CBLK
    conv = pl.pallas_call(
        _conv_body,
        grid=(grid,),
        in_specs=[pl.BlockSpec((16, CBLK), lambda m: (0, m))],
        out_specs=pl.BlockSpec((CBLK // 8, 128), lambda m: (m, 0)),
        out_shape=jax.ShapeDtypeStruct((NROWS * 16 // 128, 128), jnp.float32),
    )(t)
    return conv.reshape(NROWS, D)


def _sc_reduce_body(u_table, i_table, w_u, w_i, u_idx, i_idx, u_widx, i_widx,
                    u_sum, u_sq, u_w, i_sum, i_sq, i_w,
                    idx_v, widx_v, emb_v, wg_v, sum_v, sq_v, wsum_v,
                    sem_e, sem_w):
    wid = lax.axis_index("s") * NC + lax.axis_index("c")
    base = wid * RPW

    for (table, wtab, idx_hbm, widx_hbm, o_sum, o_sq, o_w) in (
        (u_table, w_u, u_idx, u_widx, u_sum, u_sq, u_w),
        (i_table, w_i, i_idx, i_widx, i_sum, i_sq, i_w),
    ):
        pltpu.sync_copy(idx_hbm.at[pl.ds(wid * RPW * F, RPW * F)], idx_v)
        pltpu.sync_copy(widx_hbm.at[pl.ds(wid * RPW * F, RPW * F)], widx_v)
        copies = []
        for j in range(NCHUNK):
            copies.append(pltpu.async_copy(
                table.at[idx_v.at[pl.ds(j * CHUNK, CHUNK)]],
                emb_v.at[pl.ds(j * CHUNK, CHUNK)], sem_e))
            copies.append(pltpu.async_copy(
                wtab.at[widx_v.at[pl.ds(j * CHUNK, CHUNK)]],
                wg_v.at[pl.ds(j * CHUNK, CHUNK)], sem_w))
        for c in copies:
            c.wait()

        def row_body(b, _):
            k0 = b * F
            acc = emb_v[k0, :]
            sq = acc * acc
            for f in range(1, F):
                v = emb_v[k0 + f, :]
                acc = acc + v
                sq = sq + v * v
            sum_v[b, :] = acc
            sq_v[b, :] = sq
            return 0

        lax.fori_loop(0, RPW, row_body, 0, unroll=False)

        def w_body(g, _):
            k0 = g * (16 * F)
            acc = wg_v[pl.ds(k0, 16)]
            for f in range(1, F):
                acc = acc + wg_v[pl.ds(k0 + f * 16, 16)]
            wsum_v[pl.ds(g * 16, 16)] = acc
            return 0

        lax.fori_loop(0, RPW // 16, w_body, 0, unroll=False)

        pltpu.sync_copy(sum_v, o_sum.at[pl.ds(base, RPW)])
        pltpu.sync_copy(sq_v, o_sq.at[pl.ds(base, RPW)])
        pltpu.sync_copy(wsum_v, o_w.at[pl.ds(base, RPW)])


def _sc_pair_body(u_sum, u_sq, u_w, i_sum, i_sq, i_w, a_idx, b_idx,
                  us_o, uq_o, uw_o, is_o, iq_o, iw_o,
                  av, bv, rsum, rsq, rw, sem):
    wid = lax.axis_index("s") * NC + lax.axis_index("c")
    base = wid * RPW
    pltpu.sync_copy(a_idx.at[pl.ds(base, RPW)], av)
    pltpu.sync_copy(b_idx.at[pl.ds(base, RPW)], bv)
    for (src_sum, src_sq, src_w, iv, o_sum, o_sq, o_w) in (
        (u_sum, u_sq, u_w, av, us_o, uq_o, uw_o),
        (i_sum, i_sq, i_w, bv, is_o, iq_o, iw_o),
    ):
        pltpu.async_copy(src_sum.at[iv], rsum, sem).wait()
        pltpu.sync_copy(rsum, o_sum.at[pl.ds(base, RPW)])
        pltpu.async_copy(src_sq.at[iv], rsq, sem).wait()
        pltpu.sync_copy(rsq, o_sq.at[pl.ds(base, RPW)])
        pltpu.async_copy(src_w.at[iv], rw, sem).wait()
        pltpu.sync_copy(rw, o_w.at[pl.ds(base, RPW)])


def _tc_final_body(usum_ref, isum_ref, us_ref, uq_ref, uw_ref,
                   is_ref, iq_ref, iw_ref, h1_ref, h2_ref, gb_ref, out_ref):
    P = usum_ref[...]
    Q = isum_ref[...]
    dn = (((0,), (0,)), ((), ()))
    PtP = lax.dot_general(P, P, dn, precision=lax.Precision.HIGHEST,
                          preferred_element_type=jnp.float32)
    QtQ = lax.dot_general(Q, Q, dn, precision=lax.Precision.HIGHEST,
                          preferred_element_type=jnp.float32)
    h2r = h2_ref[...]
    M = PtP * QtQ
    t = lax.dot_general(h2r, M, (((1,), (0,)), ((), ())),
                        precision=lax.Precision.HIGHEST,
                        preferred_element_type=jnp.float32)
    whole = NEG_W * jnp.sum(t * h2r)

    h1r = h1_ref[...]
    us = us_ref[...]
    vs = is_ref[...]
    bi_u = 0.5 * (us * us - uq_ref[...])
    bi_v = 0.5 * (vs * vs - iq_ref[...])
    s_u = jnp.sum(bi_u * h1r, axis=1, keepdims=True) + uw_ref[...] + gb_ref[0, 0]
    s_v = jnp.sum(bi_v * h1r, axis=1, keepdims=True) + iw_ref[...]
    y = jnp.sum(us * vs * h2r, axis=1, keepdims=True) + s_u + s_v
    pair = jnp.sum((1.0 - NEG_W) * y * y - 2.0 * y)
    out_ref[...] = jnp.reshape(whole + pair, (1, 1))


def kernel(user_features_embeddings, item_features_embeddings, w_user, w_item,
           global_bias, h1, h2, user_feature_indexes, item_feature_indexes,
           positive_pairs):
    f32 = jnp.float32
    u_idx2 = user_feature_indexes.astype(jnp.int32).reshape(-1)
    i_idx2 = item_feature_indexes.astype(jnp.int32).reshape(-1)
    u_widx = (user_feature_indexes.astype(jnp.int32)
              .reshape(BROWS // 16, 16, F).transpose(0, 2, 1).reshape(-1))
    i_widx = (item_feature_indexes.astype(jnp.int32)
              .reshape(BROWS // 16, 16, F).transpose(0, 2, 1).reshape(-1))
    a_idx = positive_pairs[:, 0].astype(jnp.int32)
    b_idx = positive_pairs[:, 1].astype(jnp.int32)

    conv_u = _relayout(user_features_embeddings)
    conv_i = _relayout(item_features_embeddings)

    mesh = plsc.VectorSubcoreMesh(core_axis_name="c", subcore_axis_name="s")
    stat = jax.ShapeDtypeStruct((BROWS, D), f32)
    vec = jax.ShapeDtypeStruct((BROWS,), f32)

    sc_params = pltpu.CompilerParams(use_tc_tiling_on_sc=False)
    reduce_fn = pl.kernel(
        _sc_reduce_body,
        out_type=(stat, stat, vec, stat, stat, vec),
        mesh=mesh,
        compiler_params=sc_params,
        scratch_types=[
            pltpu.VMEM((RPW * F,), jnp.int32),
            pltpu.VMEM((RPW * F,), jnp.int32),
            pltpu.VMEM((RPW * F, D), f32),
            pltpu.VMEM((RPW * F,), f32),
            pltpu.VMEM((RPW, D), f32),
            pltpu.VMEM((RPW, D), f32),
            pltpu.VMEM((RPW,), f32),
            pltpu.SemaphoreType.DMA,
            pltpu.SemaphoreType.DMA,
        ],
    )
    u_sum, u_sq, u_w, i_sum, i_sq, i_w = reduce_fn(
        conv_u, conv_i, w_user, w_item, u_idx2, i_idx2, u_widx, i_widx)

    pair_fn = pl.kernel(
        _sc_pair_body,
        out_type=(stat, stat, vec, stat, stat, vec),
        mesh=mesh,
        compiler_params=sc_params,
        scratch_types=[
            pltpu.VMEM((RPW,), jnp.int32),
            pltpu.VMEM((RPW,), jnp.int32),
            pltpu.VMEM((RPW, D), f32),
            pltpu.VMEM((RPW, D), f32),
            pltpu.VMEM((RPW,), f32),
            pltpu.SemaphoreType.DMA,
        ],
    )
    us, uq, uw, is_, iq, iw = pair_fn(u_sum, u_sq, u_w, i_sum, i_sq, i_w,
                                      a_idx, b_idx)

    loss = pl.pallas_call(
        _tc_final_body,
        out_shape=jax.ShapeDtypeStruct((1, 1), f32),
    )(u_sum, i_sum, us, uq, uw.reshape(BROWS, 1),
      is_, iq, iw.reshape(BROWS, 1),
      h1.reshape(1, D), h2.reshape(1, D), global_bias.reshape(1, 1))
    return loss.reshape(1)


# speedup vs baseline: 1.1484x; 1.1340x over previous
import jax
import jax.numpy as jnp
from jax import lax
from jax.experimental import pallas as pl
from jax.experimental.pallas import tpu as pltpu
from jax.experimental.pallas import tpu_sc as plsc

D = 16
F = 26
BROWS = 4096
NROWS = 1000000
NC, NS = 2, 16
NW = NC * NS
RPW = BROWS // NW
CHUNK = 128
NCHUNK = RPW * F // CHUNK
NEG_W = 0.5
CBLK = 8192


def _conv_body(x_ref, o_ref):
    x = x_ref[...]
    y = jnp.transpose(x, (1, 0))
    y3 = y.reshape(CBLK // 8, 8, 16)
    for t in range(8):
        o_ref[:, 16 * t:16 * (t + 1)] = y3[:, t, :]


def _relayout(table):
    t = table.T
    grid = (NROWS + CBLK - 1) // CBLK
    conv = pl.pallas_call(
        _conv_body,
        grid=(grid,),
        in_specs=[pl.BlockSpec((16, CBLK), lambda m: (0, m))],
        out_specs=pl.BlockSpec((CBLK // 8, 128), lambda m: (m, 0)),
        out_shape=jax.ShapeDtypeStruct((NROWS * 16 // 128, 128), jnp.float32),
    )(t)
    return conv.reshape(NROWS, D)


def _sc_reduce_body(u_table, i_table, w_u, w_i, u_idx, i_idx, u_widx, i_widx,
                    u_sum, u_sq, u_w, i_sum, i_sq, i_w,
                    idx_v, widx_v, emb_v, wg_v, sum_v, sq_v, wsum_v,
                    sem_e, sem_w):
    wid = lax.axis_index("s") * NC + lax.axis_index("c")
    base = wid * RPW

    for (table, wtab, idx_hbm, widx_hbm, o_sum, o_sq, o_w) in (
        (u_table, w_u, u_idx, u_widx, u_sum, u_sq, u_w),
        (i_table, w_i, i_idx, i_widx, i_sum, i_sq, i_w),
    ):
        pltpu.sync_copy(idx_hbm.at[pl.ds(wid * RPW * F, RPW * F)], idx_v)
        pltpu.sync_copy(widx_hbm.at[pl.ds(wid * RPW * F, RPW * F)], widx_v)
        copies = []
        for j in range(NCHUNK):
            copies.append(pltpu.async_copy(
                table.at[idx_v.at[pl.ds(j * CHUNK, CHUNK)]],
                emb_v.at[pl.ds(j * CHUNK, CHUNK)], sem_e))
            copies.append(pltpu.async_copy(
                wtab.at[widx_v.at[pl.ds(j * CHUNK, CHUNK)]],
                wg_v.at[pl.ds(j * CHUNK, CHUNK)], sem_w))
        for c in copies:
            c.wait()

        def row_body(b, _):
            k0 = b * F
            acc = emb_v[k0, :]
            sq = acc * acc
            for f in range(1, F):
                v = emb_v[k0 + f, :]
                acc = acc + v
                sq = sq + v * v
            sum_v[b, :] = acc
            sq_v[b, :] = sq
            return 0

        lax.fori_loop(0, RPW, row_body, 0, unroll=False)

        def w_body(g, _):
            k0 = g * (16 * F)
            acc = wg_v[pl.ds(k0, 16)]
            for f in range(1, F):
                acc = acc + wg_v[pl.ds(k0 + f * 16, 16)]
            wsum_v[pl.ds(g * 16, 16)] = acc
            return 0

        lax.fori_loop(0, RPW // 16, w_body, 0, unroll=False)

        pltpu.sync_copy(sum_v, o_sum.at[pl.ds(base, RPW)])
        pltpu.sync_copy(sq_v, o_sq.at[pl.ds(base, RPW)])
        pltpu.sync_copy(wsum_v, o_w.at[pl.ds(base, RPW)])


def _sc_pair_body(u_sum, u_sq, u_w, i_sum, i_sq, i_w, a_idx, b_idx,
                  us_o, uq_o, uw_o, is_o, iq_o, iw_o,
                  av, bv, rsum, rsq, rw, sem):
    wid = lax.axis_index("s") * NC + lax.axis_index("c")
    base = wid * RPW
    pltpu.sync_copy(a_idx.at[pl.ds(base, RPW)], av)
    pltpu.sync_copy(b_idx.at[pl.ds(base, RPW)], bv)
    for (src_sum, src_sq, src_w, iv, o_sum, o_sq, o_w) in (
        (u_sum, u_sq, u_w, av, us_o, uq_o, uw_o),
        (i_sum, i_sq, i_w, bv, is_o, iq_o, iw_o),
    ):
        pltpu.async_copy(src_sum.at[iv], rsum, sem).wait()
        pltpu.sync_copy(rsum, o_sum.at[pl.ds(base, RPW)])
        pltpu.async_copy(src_sq.at[iv], rsq, sem).wait()
        pltpu.sync_copy(rsq, o_sq.at[pl.ds(base, RPW)])
        pltpu.async_copy(src_w.at[iv], rw, sem).wait()
        pltpu.sync_copy(rw, o_w.at[pl.ds(base, RPW)])


def _tc_final_body(usum_ref, isum_ref, us_ref, uq_ref, uw_ref,
                   is_ref, iq_ref, iw_ref, h1_ref, h2_ref, gb_ref, out_ref):
    P = usum_ref[...]
    Q = isum_ref[...]
    dn = (((0,), (0,)), ((), ()))
    PtP = lax.dot_general(P, P, dn, precision=lax.Precision.HIGHEST,
                          preferred_element_type=jnp.float32)
    QtQ = lax.dot_general(Q, Q, dn, precision=lax.Precision.HIGHEST,
                          preferred_element_type=jnp.float32)
    h2r = h2_ref[...]
    M = PtP * QtQ
    t = lax.dot_general(h2r, M, (((1,), (0,)), ((), ())),
                        precision=lax.Precision.HIGHEST,
                        preferred_element_type=jnp.float32)
    whole = NEG_W * jnp.sum(t * h2r)

    h1r = h1_ref[...]
    us = us_ref[...]
    vs = is_ref[...]
    bi_u = 0.5 * (us * us - uq_ref[...])
    bi_v = 0.5 * (vs * vs - iq_ref[...])
    s_u = jnp.sum(bi_u * h1r, axis=1, keepdims=True) + uw_ref[...] + gb_ref[0, 0]
    s_v = jnp.sum(bi_v * h1r, axis=1, keepdims=True) + iw_ref[...]
    y = jnp.sum(us * vs * h2r, axis=1, keepdims=True) + s_u + s_v
    pair = jnp.sum((1.0 - NEG_W) * y * y - 2.0 * y)
    out_ref[...] = jnp.reshape(whole + pair, (1, 1))


def kernel(user_features_embeddings, item_features_embeddings, w_user, w_item,
           global_bias, h1, h2, user_feature_indexes, item_feature_indexes,
           positive_pairs):
    f32 = jnp.float32
    u_idx2 = user_feature_indexes.astype(jnp.int32).reshape(-1)
    i_idx2 = item_feature_indexes.astype(jnp.int32).reshape(-1)
    u_widx = (user_feature_indexes.astype(jnp.int32)
              .reshape(BROWS // 16, 16, F).transpose(0, 2, 1).reshape(-1))
    i_widx = (item_feature_indexes.astype(jnp.int32)
              .reshape(BROWS // 16, 16, F).transpose(0, 2, 1).reshape(-1))
    a_idx = positive_pairs[:, 0].astype(jnp.int32)
    b_idx = positive_pairs[:, 1].astype(jnp.int32)

    conv_u = _relayout(user_features_embeddings)
    conv_i = _relayout(item_features_embeddings)

    mesh = plsc.VectorSubcoreMesh(core_axis_name="c", subcore_axis_name="s")
    stat = jax.ShapeDtypeStruct((BROWS, D), f32)
    vec = jax.ShapeDtypeStruct((BROWS,), f32)

    sc_params = pltpu.CompilerParams(use_tc_tiling_on_sc=False)
    reduce_fn = pl.kernel(
        _sc_reduce_body,
        out_type=(stat, stat, vec, stat, stat, vec),
        mesh=mesh,
        compiler_params=sc_params,
        scratch_types=[
            pltpu.VMEM((RPW * F,), jnp.int32),
            pltpu.VMEM((RPW * F,), jnp.int32),
            pltpu.VMEM((RPW * F, D), f32),
            pltpu.VMEM((RPW * F,), f32),
            pltpu.VMEM((RPW, D), f32),
            pltpu.VMEM((RPW, D), f32),
            pltpu.VMEM((RPW,), f32),
            pltpu.SemaphoreType.DMA,
            pltpu.SemaphoreType.DMA,
        ],
    )
    u_sum, u_sq, u_w, i_sum, i_sq, i_w = reduce_fn(
        conv_u, conv_i, w_user, w_item, u_idx2, i_idx2, u_widx, i_widx)

    pair_fn = pl.kernel(
        _sc_pair_body,
        out_type=(stat, stat, vec, stat, stat, vec),
        mesh=mesh,
        compiler_params=sc_params,
        scratch_types=[
            pltpu.VMEM((RPW,), jnp.int32),
            pltpu.VMEM((RPW,), jnp.int32),
            pltpu.VMEM((RPW, D), f32),
            pltpu.VMEM((RPW, D), f32),
            pltpu.VMEM((RPW,), f32),
            pltpu.SemaphoreType.DMA,
        ],
    )
    us, uq, uw, is_, iq, iw = pair_fn(u_sum, u_sq, u_w, i_sum, i_sq, i_w,
                                      a_idx, b_idx)

    loss = pl.pallas_call(
        _tc_final_body,
        out_shape=jax.ShapeDtypeStruct((1, 1), f32),
    )(u_sum, i_sum, us, uq, uw.reshape(BROWS, 1),
      is_, iq, iw.reshape(BROWS, 1),
      h1.reshape(1, D), h2.reshape(1, D), global_bias.reshape(1, 1))
    return loss.reshape(1)
